# baseline (speedup 1.0000x reference)
import functools

import jax
import jax.numpy as jnp
from jax import lax
from jax.experimental import pallas as pl
from jax.experimental.pallas import tpu as pltpu
from jax.experimental.pallas import tpu_sc as plsc

_C, _L, _IN, _N = 4096, 128, 173, 8192
_TN = 512
_NT = _N // _TN
_MT = _C // _TN


def _mm_body(thetas_ref, w_in_ref, b_ref, wv_ref, m_ref, w_ref, idx_ref, vcat_ref):
    i = pl.program_id(0)

    @pl.when(i < _NT)
    def _compute():
        th = thetas_ref[...]
        wblk = lax.dot_general(th, w_in_ref[...], (((0,), (1,)), ((), ())))
        wblk = wblk + b_ref[...]
        w_ref[...] = wblk
        mx = jnp.max(wblk, axis=1, keepdims=True)
        cols = lax.broadcasted_iota(jnp.int32, (_TN, _C), 1)
        amin = jnp.min(jnp.where(wblk == mx, cols, _C), axis=1)
        idx_ref[...] = amin.reshape(1, _TN)
        vcat_ref[...] = lax.dot_general(th, wv_ref[...], (((0,), (0,)), ((), ())))

    @pl.when(i >= _NT)
    def _copy_m():
        vcat_ref[...] = m_ref[...]


def _fused_mm(thetas, W, b2, Wv, M):
    return pl.pallas_call(
        _mm_body,
        grid=(_NT + _MT,),
        in_specs=[
            pl.BlockSpec((_IN, _TN), lambda i: (0, jnp.minimum(i, _NT - 1))),
            pl.BlockSpec((_C, _IN), lambda i: (0, 0)),
            pl.BlockSpec((1, _C), lambda i: (0, 0)),
            pl.BlockSpec((_IN, _L), lambda i: (0, 0)),
            pl.BlockSpec((_TN, _L), lambda i: (jnp.maximum(i - _NT, 0), 0)),
        ],
        out_specs=[
            pl.BlockSpec((_TN, _C), lambda i: (jnp.minimum(i, _NT - 1), 0)),
            pl.BlockSpec((1, _TN), lambda i: (0, jnp.minimum(i, _NT - 1))),
            pl.BlockSpec((_TN, _L), lambda i: (i, 0)),
        ],
        out_shape=[
            jax.ShapeDtypeStruct((_N, _C), jnp.float32),
            jax.ShapeDtypeStruct((1, _N), jnp.int32),
            jax.ShapeDtypeStruct((_N + _C, _L), jnp.float32),
        ],
    )(thetas, W, b2, Wv, M)


_NSUB = 16
_NPW = _N // _NSUB
_CCHUNK = _C // (2 * _NSUB)


def _sc_scatter_body(idx_hbm, vcat_hbm, out_hbm,
                     idx_v, last_v, shared_sp, mrg_v, src_v, rows_v, sem):
    cid = lax.axis_index("c")
    sid = lax.axis_index("s")
    ii = lax.iota(jnp.int32, 16)

    nbase = sid * _NPW
    pltpu.sync_copy(idx_hbm.at[pl.ds(nbase, _NPW)], idx_v)

    def _init(i, _):
        last_v[pl.ds(i * 16, 16)] = jnp.full((16,), -1, jnp.int32)
        return 0

    lax.fori_loop(0, _C // 16, _init, 0)

    for t in range(_NPW // 16):
        c16 = idx_v[pl.ds(t * 16, 16)]
        key = c16 * 8192 + (nbase + t * 16 + ii)
        ks = lax.sort(key)
        cs = lax.shift_right_logical(ks, 13)
        ns = lax.bitwise_and(ks, 8191)
        nxt = cs.at[jnp.minimum(ii + 1, 15)].get(mode="promise_in_bounds")
        winner = (cs != nxt) | (ii == 15)
        plsc.store_scatter(last_v, [cs], ns, mask=winner)

    pltpu.sync_copy(last_v, shared_sp.at[sid])
    plsc.subcore_barrier()

    base = sid * (2 * _CCHUNK) + cid * _CCHUNK
    for wkr in range(_NSUB):
        pltpu.sync_copy(shared_sp.at[wkr, pl.ds(base, _CCHUNK)], mrg_v.at[wkr])
    for j in range(_CCHUNK // 16):
        acc = mrg_v[0, pl.ds(j * 16, 16)]
        for wkr in range(1, _NSUB):
            acc = jnp.maximum(acc, mrg_v[wkr, pl.ds(j * 16, 16)])
        slot = base + j * 16 + ii
        src_v[pl.ds(j * 16, 16)] = jnp.where(acc >= 0, acc, _N + slot)

    pltpu.async_copy(vcat_hbm.at[src_v], rows_v, sem).wait()
    pltpu.sync_copy(rows_v, out_hbm.at[pl.ds(base, _CCHUNK)])


@functools.partial(
    pl.kernel,
    mesh=plsc.VectorSubcoreMesh(core_axis_name="c", subcore_axis_name="s"),
    compiler_params=pltpu.CompilerParams(needs_layout_passes=False),
    out_type=jax.ShapeDtypeStruct((_C, _L), jnp.float32),
    scratch_types=[
        pltpu.VMEM((_NPW,), jnp.int32),
        pltpu.VMEM((_C,), jnp.int32),
        pltpu.VMEM_SHARED((_NSUB, _C), jnp.int32),
        pltpu.VMEM((_NSUB, _CCHUNK), jnp.int32),
        pltpu.VMEM((_CCHUNK,), jnp.int32),
        pltpu.VMEM((_CCHUNK, _L), jnp.float32),
        pltpu.SemaphoreType.DMA,
    ],
)
def _sc_scatter(idx_hbm, vcat_hbm, out_hbm,
                idx_v, last_v, shared_sp, mrg_v, src_v, rows_v, sem):
    _sc_scatter_body(idx_hbm, vcat_hbm, out_hbm,
                     idx_v, last_v, shared_sp, mrg_v, src_v, rows_v, sem)


def kernel(thetas, W, b, M, Wv):
    w, idx2, vcat = _fused_mm(thetas, W, b.reshape(1, _C), Wv, M)
    M_new = _sc_scatter(idx2.reshape(_N), vcat)
    return (w, M_new)


# speedup vs baseline: 1.0386x; 1.0386x over previous
import functools

import jax
import jax.numpy as jnp
from jax import lax
from jax.experimental import pallas as pl
from jax.experimental.pallas import tpu as pltpu
from jax.experimental.pallas import tpu_sc as plsc

_C, _L, _IN, _N = 4096, 128, 173, 8192
_TN = 512
_NT = _N // _TN
_MT = _C // _TN


def _mm_body(thetas_ref, wt_ref, b_ref, m_ref, w_ref, idx_ref, vcat_ref):
    i = pl.program_id(0)

    @pl.when(i < _NT)
    def _compute():
        th = thetas_ref[...]
        wv = wt_ref[...]
        wblk = lax.dot_general(th, wv[:, :_C], (((0,), (0,)), ((), ())))
        wblk = wblk + b_ref[...]
        w_ref[...] = wblk
        mx = jnp.max(wblk, axis=1, keepdims=True)
        cols = lax.broadcasted_iota(jnp.int32, (_TN, _C), 1)
        amin = jnp.min(jnp.where(wblk == mx, cols, _C), axis=1)
        idx_ref[...] = amin.reshape(1, _TN)
        vcat_ref[...] = lax.dot_general(th, wv[:, _C:], (((0,), (0,)), ((), ())))

    @pl.when(i >= _NT)
    def _copy_m():
        vcat_ref[...] = m_ref[...]


def _fused_mm(thetas, wtv, b2, M):
    return pl.pallas_call(
        _mm_body,
        grid=(_NT + _MT,),
        in_specs=[
            pl.BlockSpec((_IN, _TN), lambda i: (0, jnp.minimum(i, _NT - 1))),
            pl.BlockSpec((_IN, _C + _L), lambda i: (0, 0)),
            pl.BlockSpec((1, _C), lambda i: (0, 0)),
            pl.BlockSpec((_TN, _L), lambda i: (jnp.maximum(i - _NT, 0), 0)),
        ],
        out_specs=[
            pl.BlockSpec((_TN, _C), lambda i: (jnp.minimum(i, _NT - 1), 0)),
            pl.BlockSpec((1, _TN), lambda i: (0, jnp.minimum(i, _NT - 1))),
            pl.BlockSpec((_TN, _L), lambda i: (i, 0)),
        ],
        out_shape=[
            jax.ShapeDtypeStruct((_N, _C), jnp.float32),
            jax.ShapeDtypeStruct((1, _N), jnp.int32),
            jax.ShapeDtypeStruct((_N + _C, _L), jnp.float32),
        ],
    )(thetas, wtv, b2, M)


_NSUB = 16
_NPW = _N // _NSUB
_CCHUNK = _C // (2 * _NSUB)


def _sc_scatter_body(idx_hbm, vcat_hbm, out_hbm,
                     idx_v, last_v, shared_sp, mrg_v, src_v, rows_v, sem):
    cid = lax.axis_index("c")
    sid = lax.axis_index("s")
    ii = lax.iota(jnp.int32, 16)

    nbase = sid * _NPW
    pltpu.sync_copy(idx_hbm.at[pl.ds(nbase, _NPW)], idx_v)

    def _init(i, _):
        last_v[pl.ds(i * 16, 16)] = jnp.full((16,), -1, jnp.int32)
        return 0

    lax.fori_loop(0, _C // 16, _init, 0)

    for t in range(_NPW // 16):
        c16 = idx_v[pl.ds(t * 16, 16)]
        key = c16 * 8192 + (nbase + t * 16 + ii)
        ks = lax.sort(key)
        cs = lax.shift_right_logical(ks, 13)
        ns = lax.bitwise_and(ks, 8191)
        nxt = cs.at[jnp.minimum(ii + 1, 15)].get(mode="promise_in_bounds")
        winner = (cs != nxt) | (ii == 15)
        plsc.store_scatter(last_v, [cs], ns, mask=winner)

    pltpu.sync_copy(last_v, shared_sp.at[sid])
    plsc.subcore_barrier()

    base = sid * (2 * _CCHUNK) + cid * _CCHUNK
    pltpu.sync_copy(shared_sp.at[:, pl.ds(base, _CCHUNK)], mrg_v)
    for j in range(_CCHUNK // 16):
        acc = mrg_v[0, pl.ds(j * 16, 16)]
        for wkr in range(1, _NSUB):
            acc = jnp.maximum(acc, mrg_v[wkr, pl.ds(j * 16, 16)])
        slot = base + j * 16 + ii
        src_v[pl.ds(j * 16, 16)] = jnp.where(acc >= 0, acc, _N + slot)

    pltpu.async_copy(vcat_hbm.at[src_v], rows_v, sem).wait()
    pltpu.sync_copy(rows_v, out_hbm.at[pl.ds(base, _CCHUNK)])


@functools.partial(
    pl.kernel,
    mesh=plsc.VectorSubcoreMesh(core_axis_name="c", subcore_axis_name="s"),
    compiler_params=pltpu.CompilerParams(needs_layout_passes=False),
    out_type=jax.ShapeDtypeStruct((_C, _L), jnp.float32),
    scratch_types=[
        pltpu.VMEM((_NPW,), jnp.int32),
        pltpu.VMEM((_C,), jnp.int32),
        pltpu.VMEM_SHARED((_NSUB, _C), jnp.int32),
        pltpu.VMEM((_NSUB, _CCHUNK), jnp.int32),
        pltpu.VMEM((_CCHUNK,), jnp.int32),
        pltpu.VMEM((_CCHUNK, _L), jnp.float32),
        pltpu.SemaphoreType.DMA,
    ],
)
def _sc_scatter(idx_hbm, vcat_hbm, out_hbm,
                idx_v, last_v, shared_sp, mrg_v, src_v, rows_v, sem):
    _sc_scatter_body(idx_hbm, vcat_hbm, out_hbm,
                     idx_v, last_v, shared_sp, mrg_v, src_v, rows_v, sem)


def kernel(thetas, W, b, M, Wv):
    wtv = jnp.concatenate([W.T, Wv], axis=1)
    w, idx2, vcat = _fused_mm(thetas, wtv, b.reshape(1, _C), M)
    M_new = _sc_scatter(idx2.reshape(_N), vcat)
    return (w, M_new)


# speedup vs baseline: 1.0837x; 1.0434x over previous
import functools

import jax
import jax.numpy as jnp
from jax import lax
from jax.experimental import pallas as pl
from jax.experimental.pallas import tpu as pltpu
from jax.experimental.pallas import tpu_sc as plsc

_C, _L, _IN, _N = 4096, 128, 173, 8192
_TN = 1024
_NT = _N // _TN
_MT = _C // _TN


def _mm_body(thetas_ref, wt_ref, b_ref, m_ref, w_ref, idx_ref, vcat_ref):
    i = pl.program_id(0)

    @pl.when(i < _NT)
    def _compute():
        th = thetas_ref[...]
        wv = wt_ref[...]
        wblk = lax.dot_general(th, wv[:, :_C], (((0,), (0,)), ((), ())))
        wblk = wblk + b_ref[...]
        w_ref[...] = wblk
        mx = jnp.max(wblk, axis=1, keepdims=True)
        cols = lax.broadcasted_iota(jnp.int32, (_TN, _C), 1)
        amin = jnp.min(jnp.where(wblk == mx, cols, _C), axis=1)
        idx_ref[...] = amin.reshape(1, _TN)
        vcat_ref[...] = lax.dot_general(th, wv[:, _C:], (((0,), (0,)), ((), ())))

    @pl.when(i >= _NT)
    def _copy_m():
        vcat_ref[...] = m_ref[...]


def _fused_mm(thetas, wtv, b2, M):
    return pl.pallas_call(
        _mm_body,
        grid=(_NT + _MT,),
        in_specs=[
            pl.BlockSpec((_IN, _TN), lambda i: (0, jnp.minimum(i, _NT - 1))),
            pl.BlockSpec((_IN, _C + _L), lambda i: (0, 0)),
            pl.BlockSpec((1, _C), lambda i: (0, 0)),
            pl.BlockSpec((_TN, _L), lambda i: (jnp.maximum(i - _NT, 0), 0)),
        ],
        out_specs=[
            pl.BlockSpec((_TN, _C), lambda i: (jnp.minimum(i, _NT - 1), 0)),
            pl.BlockSpec((1, _TN), lambda i: (0, jnp.minimum(i, _NT - 1))),
            pl.BlockSpec((_TN, _L), lambda i: (i, 0)),
        ],
        out_shape=[
            jax.ShapeDtypeStruct((_N, _C), jnp.float32),
            jax.ShapeDtypeStruct((1, _N), jnp.int32),
            jax.ShapeDtypeStruct((_N + _C, _L), jnp.float32),
        ],
    )(thetas, wtv, b2, M)


_NSUB = 16
_NPW = _N // _NSUB
_CCHUNK = _C // (2 * _NSUB)


def _sc_scatter_body(idx_hbm, vcat_hbm, out_hbm,
                     idx_v, last_v, shared_sp, mrg_v, src_v, rows_v, sem):
    cid = lax.axis_index("c")
    sid = lax.axis_index("s")
    ii = lax.iota(jnp.int32, 16)

    nbase = sid * _NPW
    pltpu.sync_copy(idx_hbm.at[pl.ds(nbase, _NPW)], idx_v)

    def _init(i, _):
        last_v[pl.ds(i * 16, 16)] = jnp.full((16,), -1, jnp.int32)
        return 0

    lax.fori_loop(0, _C // 16, _init, 0)

    for t in range(_NPW // 16):
        c16 = idx_v[pl.ds(t * 16, 16)]
        key = c16 * 8192 + (nbase + t * 16 + ii)
        ks = lax.sort(key)
        cs = lax.shift_right_logical(ks, 13)
        ns = lax.bitwise_and(ks, 8191)
        nxt = cs.at[jnp.minimum(ii + 1, 15)].get(mode="promise_in_bounds")
        winner = (cs != nxt) | (ii == 15)
        plsc.store_scatter(last_v, [cs], ns, mask=winner)

    pltpu.sync_copy(last_v, shared_sp.at[sid])
    plsc.subcore_barrier()

    base = sid * (2 * _CCHUNK) + cid * _CCHUNK
    pltpu.sync_copy(shared_sp.at[:, pl.ds(base, _CCHUNK)], mrg_v)
    for j in range(_CCHUNK // 16):
        acc = mrg_v[0, pl.ds(j * 16, 16)]
        for wkr in range(1, _NSUB):
            acc = jnp.maximum(acc, mrg_v[wkr, pl.ds(j * 16, 16)])
        slot = base + j * 16 + ii
        src_v[pl.ds(j * 16, 16)] = jnp.where(acc >= 0, acc, _N + slot)

    pltpu.async_copy(vcat_hbm.at[src_v], rows_v, sem).wait()
    pltpu.sync_copy(rows_v, out_hbm.at[pl.ds(base, _CCHUNK)])


@functools.partial(
    pl.kernel,
    mesh=plsc.VectorSubcoreMesh(core_axis_name="c", subcore_axis_name="s"),
    compiler_params=pltpu.CompilerParams(needs_layout_passes=False),
    out_type=jax.ShapeDtypeStruct((_C, _L), jnp.float32),
    scratch_types=[
        pltpu.VMEM((_NPW,), jnp.int32),
        pltpu.VMEM((_C,), jnp.int32),
        pltpu.VMEM_SHARED((_NSUB, _C), jnp.int32),
        pltpu.VMEM((_NSUB, _CCHUNK), jnp.int32),
        pltpu.VMEM((_CCHUNK,), jnp.int32),
        pltpu.VMEM((_CCHUNK, _L), jnp.float32),
        pltpu.SemaphoreType.DMA,
    ],
)
def _sc_scatter(idx_hbm, vcat_hbm, out_hbm,
                idx_v, last_v, shared_sp, mrg_v, src_v, rows_v, sem):
    _sc_scatter_body(idx_hbm, vcat_hbm, out_hbm,
                     idx_v, last_v, shared_sp, mrg_v, src_v, rows_v, sem)


def kernel(thetas, W, b, M, Wv):
    wtv = jnp.concatenate([W.T, Wv], axis=1)
    w, idx2, vcat = _fused_mm(thetas, wtv, b.reshape(1, _C), M)
    M_new = _sc_scatter(idx2.reshape(_N), vcat)
    return (w, M_new)


# speedup vs baseline: 1.0967x; 1.0120x over previous
import functools

import jax
import jax.numpy as jnp
from jax import lax
from jax.experimental import pallas as pl
from jax.experimental.pallas import tpu as pltpu
from jax.experimental.pallas import tpu_sc as plsc

_C, _L, _IN, _N = 4096, 128, 173, 8192
_TN = 1024
_NT = _N // _TN
_MT = _C // _TN


def _mm_body(thetas_ref, wt_ref, b_ref, m_ref, w_ref, idx_ref, vcat_ref):
    i = pl.program_id(0)

    @pl.when(i < _NT)
    def _compute():
        th = thetas_ref[...]
        wv = wt_ref[...]
        wblk = lax.dot_general(th, wv[:, :_C], (((0,), (0,)), ((), ())))
        wblk = wblk + b_ref[...]
        w_ref[...] = wblk
        mx = jnp.max(wblk, axis=1, keepdims=True)
        cols = lax.broadcasted_iota(jnp.int32, (_TN, _C), 1)
        amin = jnp.min(jnp.where(wblk == mx, cols, _C), axis=1)
        idx_ref[...] = amin.reshape(1, _TN)
        vcat_ref[...] = lax.dot_general(th, wv[:, _C:], (((0,), (0,)), ((), ())))

    @pl.when(i >= _NT)
    def _copy_m():
        vcat_ref[...] = m_ref[...]


def _fused_mm(thetas, wtv, b2, M):
    return pl.pallas_call(
        _mm_body,
        grid=(_NT + _MT,),
        in_specs=[
            pl.BlockSpec((_IN, _TN), lambda i: (0, jnp.minimum(i, _NT - 1))),
            pl.BlockSpec((_IN, _C + _L), lambda i: (0, 0)),
            pl.BlockSpec((1, _C), lambda i: (0, 0)),
            pl.BlockSpec((_TN, _L), lambda i: (jnp.maximum(i - _NT, 0), 0)),
        ],
        out_specs=[
            pl.BlockSpec((_TN, _C), lambda i: (jnp.minimum(i, _NT - 1), 0)),
            pl.BlockSpec((1, _TN), lambda i: (0, jnp.minimum(i, _NT - 1))),
            pl.BlockSpec((_TN, _L), lambda i: (i, 0)),
        ],
        out_shape=[
            jax.ShapeDtypeStruct((_N, _C), jnp.float32),
            jax.ShapeDtypeStruct((1, _N), jnp.int32),
            jax.ShapeDtypeStruct((_N + _C, _L), jnp.float32),
        ],
    )(thetas, wtv, b2, M)


_NSUB = 16
_NPW = _N // _NSUB
_CCHUNK = _C // (2 * _NSUB)


def _sc_scatter_body(idx_hbm, vcat_hbm, out_hbm,
                     idx_v, last_v, shared_sp, mrg_v, src_v, rows_v, sem):
    cid = lax.axis_index("c")
    sid = lax.axis_index("s")
    ii = lax.iota(jnp.int32, 16)

    nbase = sid * _NPW
    pltpu.sync_copy(idx_hbm.at[pl.ds(nbase, _NPW)], idx_v)

    neg1 = jnp.full((16,), -1, jnp.int32)

    def _init(i, _):
        for u in range(8):
            last_v[pl.ds(i * 128 + u * 16, 16)] = neg1
        return 0

    lax.fori_loop(0, _C // 128, _init, 0)

    for t in range(_NPW // 16):
        c16 = idx_v[pl.ds(t * 16, 16)]
        key = c16 * 8192 + (nbase + t * 16 + ii)
        ks = lax.sort(key)
        cs = lax.shift_right_logical(ks, 13)
        ns = lax.bitwise_and(ks, 8191)
        nxt = cs.at[jnp.minimum(ii + 1, 15)].get(mode="promise_in_bounds")
        winner = (cs != nxt) | (ii == 15)
        plsc.store_scatter(last_v, [cs], ns, mask=winner)

    pltpu.sync_copy(last_v, shared_sp.at[sid])
    plsc.subcore_barrier()

    base = sid * (2 * _CCHUNK) + cid * _CCHUNK
    pltpu.sync_copy(shared_sp.at[:, pl.ds(base, _CCHUNK)], mrg_v)
    for j in range(_CCHUNK // 16):
        acc = mrg_v[0, pl.ds(j * 16, 16)]
        for wkr in range(1, _NSUB):
            acc = jnp.maximum(acc, mrg_v[wkr, pl.ds(j * 16, 16)])
        slot = base + j * 16 + ii
        src_v[pl.ds(j * 16, 16)] = jnp.where(acc >= 0, acc, _N + slot)

    pltpu.async_copy(vcat_hbm.at[src_v], rows_v, sem).wait()
    pltpu.sync_copy(rows_v, out_hbm.at[pl.ds(base, _CCHUNK)])


@functools.partial(
    pl.kernel,
    mesh=plsc.VectorSubcoreMesh(core_axis_name="c", subcore_axis_name="s"),
    compiler_params=pltpu.CompilerParams(needs_layout_passes=False),
    out_type=jax.ShapeDtypeStruct((_C, _L), jnp.float32),
    scratch_types=[
        pltpu.VMEM((_NPW,), jnp.int32),
        pltpu.VMEM((_C,), jnp.int32),
        pltpu.VMEM_SHARED((_NSUB, _C), jnp.int32),
        pltpu.VMEM((_NSUB, _CCHUNK), jnp.int32),
        pltpu.VMEM((_CCHUNK,), jnp.int32),
        pltpu.VMEM((_CCHUNK, _L), jnp.float32),
        pltpu.SemaphoreType.DMA,
    ],
)
def _sc_scatter(idx_hbm, vcat_hbm, out_hbm,
                idx_v, last_v, shared_sp, mrg_v, src_v, rows_v, sem):
    _sc_scatter_body(idx_hbm, vcat_hbm, out_hbm,
                     idx_v, last_v, shared_sp, mrg_v, src_v, rows_v, sem)


def kernel(thetas, W, b, M, Wv):
    wtv = jnp.concatenate([W.T, Wv], axis=1)
    w, idx2, vcat = _fused_mm(thetas, wtv, b.reshape(1, _C), M)
    M_new = _sc_scatter(idx2.reshape(_N), vcat)
    return (w, M_new)
